# baseline (speedup 1.0000x reference)
import functools

import jax
import jax.numpy as jnp
from jax import lax
from jax.experimental import pallas as pl
from jax.experimental.pallas import tpu as pltpu
from jax.experimental.pallas import tpu_sc as plsc

_INFO = plsc.get_sparse_core_info()
_NC, _NS = _INFO.num_cores, _INFO.num_subcores
_NW = _NC * _NS

_CHUNK = 640
_NBUF = 4


def _make_gather(n, v, d):
    assert n % _NW == 0
    per_w = n // _NW
    assert per_w % (_CHUNK * _NBUF) == 0
    n_chunks = per_w // _CHUNK
    n_groups = n_chunks // _NBUF
    mesh = plsc.VectorSubcoreMesh(core_axis_name="c", subcore_axis_name="s")

    @functools.partial(
        pl.kernel,
        mesh=mesh,
        out_type=jax.ShapeDtypeStruct((n, d), jnp.float32),
        compiler_params=pltpu.CompilerParams(use_tc_tiling_on_sc=False),
        scratch_types=[
            pltpu.VMEM((n_chunks, _CHUNK), jnp.int32),
            pltpu.VMEM((_NBUF, _CHUNK, d), jnp.float32),
        ]
        + [pltpu.SemaphoreType.DMA] * (2 * _NBUF),
    )
    def gather(idx_hbm, table_hbm, out_hbm, idx_v, rows_v, *sems):
        gsem, wsem = sems[:_NBUF], sems[_NBUF:]
        wid = lax.axis_index("s") * _NC + lax.axis_index("c")
        base = wid * per_w

        pltpu.sync_copy(idx_hbm.at[wid], idx_v)

        def start_gather(c, b):
            pltpu.async_copy(table_hbm.at[idx_v.at[c]], rows_v.at[b], gsem[b])

        def wait(sem, b):
            pltpu.make_async_copy(
                table_hbm.at[pl.ds(0, _CHUNK)], rows_v.at[b], sem
            ).wait()

        def start_write(c, b):
            pltpu.async_copy(
                rows_v.at[b], out_hbm.at[pl.ds(base + c * _CHUNK, _CHUNK)], wsem[b]
            )

        for b in range(_NBUF - 1):
            start_gather(b, b)

        def body(g, carry):
            c0 = _NBUF * g
            for b in range(_NBUF):
                c = c0 + b
                nb = (b + _NBUF - 1) % _NBUF

                if b == 0:
                    @pl.when(g >= 1)
                    def _():
                        wait(wsem[nb], nb)

                    start_gather(c + _NBUF - 1, nb)
                else:
                    @pl.when(g + 1 < n_groups)
                    def _():
                        wait(wsem[nb], nb)
                        start_gather(c + _NBUF - 1, nb)

                wait(gsem[b], b)
                start_write(c, b)
            return carry

        lax.fori_loop(0, n_groups, body, 0)
        for b in range(_NBUF):
            wait(wsem[b], b)

    return gather


def kernel(x, table):
    b, l = x.shape
    v, d = table.shape
    n = b * l
    flat = x.reshape(n).astype(jnp.int32)
    per_w = n // _NW
    idx3 = flat.reshape(_NW, per_w // _CHUNK, _CHUNK)
    out = _make_gather(n, v, d)(idx3, table)
    return out.reshape(b, l, d)


# speedup vs baseline: 1.1111x; 1.1111x over previous
import functools

import jax
import jax.numpy as jnp
from jax import lax
from jax.experimental import pallas as pl
from jax.experimental.pallas import tpu as pltpu
from jax.experimental.pallas import tpu_sc as plsc

_INFO = plsc.get_sparse_core_info()
_NC, _NS = _INFO.num_cores, _INFO.num_subcores
_NW = _NC * _NS


def _make_gather(b, l, v, d):
    assert b % (_NW * 128) == 0 and d == 32 and v % 4 == 0
    bw = b // _NW
    nbb = bw // 128
    n_units = l * nbb
    assert n_units % 2 == 0
    ltr = (l + 7) // 8
    mesh = plsc.VectorSubcoreMesh(core_axis_name="c", subcore_axis_name="s")

    @functools.partial(
        pl.kernel,
        mesh=mesh,
        out_type=jax.ShapeDtypeStruct((l, d, b), jnp.float32),
        compiler_params=pltpu.CompilerParams(
            use_tc_tiling_on_sc=True, needs_layout_passes=False
        ),
        scratch_types=[
            pltpu.VMEM((ltr, nbb, 8, 128), jnp.int32),
            pltpu.VMEM((2, 128), jnp.int32),
            pltpu.VMEM((2, 128), jnp.int32),
            pltpu.VMEM((2, 128, 128), jnp.float32),
            pltpu.VMEM((2, 4, 8, 128), jnp.float32),
            pltpu.SemaphoreType.DMA,
            pltpu.SemaphoreType.DMA,
            pltpu.SemaphoreType.DMA,
            pltpu.SemaphoreType.DMA,
        ],
    )
    def gather(xt_hbm, rtab_hbm, out_hbm, xslab, gidx, qcol, gbuf, otile, *sems):
        gsem = sems[:2]
        wsem = sems[2:]
        wid = lax.axis_index("s") * _NC + lax.axis_index("c")
        col0 = wid * bw

        for tc in range(nbb):
            for tr in range(ltr):
                rows = min(8, l - tr * 8)
                pltpu.sync_copy(
                    xt_hbm.at[pl.ds(tr * 8, rows), pl.ds(col0 + tc * 128, 128)],
                    xslab.at[tr, tc, pl.ds(0, rows)],
                )

        lane = lax.iota(jnp.int32, 16)

        def prep(u, bf):
            li = u // nbb
            tc = lax.rem(u, nbb)
            tr, r = li // 8, lax.rem(li, 8)
            for g in range(8):
                idx = xslab[tr, tc, r, pl.ds(g * 16, 16)]
                gidx[bf, pl.ds(g * 16, 16)] = idx >> 2
                qcol[bf, pl.ds(g * 16, 16)] = (idx & 3) * d

        def start_gather(bf):
            pltpu.async_copy(rtab_hbm.at[gidx.at[bf]], gbuf.at[bf], gsem[bf])

        def wait_gather(bf):
            pltpu.make_async_copy(
                rtab_hbm.at[pl.ds(0, 128)], gbuf.at[bf], gsem[bf]
            ).wait()

        def build(u, bf):
            def tile_row(db, carry):
                for d8 in range(8):
                    for g in range(8):
                        rows = lane + g * 16
                        cols = qcol[bf, pl.ds(g * 16, 16)] + (db * 8 + d8)
                        otile[bf, db, d8, pl.ds(g * 16, 16)] = plsc.load_gather(
                            gbuf.at[bf], [rows, cols]
                        )
                return carry

            lax.fori_loop(0, 4, tile_row, 0)

        def start_write(u, bf):
            li = u // nbb
            tc = lax.rem(u, nbb)
            for db in range(4):
                pltpu.async_copy(
                    otile.at[bf, db],
                    out_hbm.at[li, pl.ds(db * 8, 8), pl.ds(col0 + tc * 128, 128)],
                    wsem[bf],
                )

        def wait_writes(bf):
            for _db in range(4):
                pltpu.make_async_copy(
                    out_hbm.at[0, pl.ds(0, 8), pl.ds(0, 128)],
                    otile.at[bf, _db],
                    wsem[bf],
                ).wait()

        prep(0, 0)
        start_gather(0)

        def body(g, carry):
            u0 = 2 * g

            prep(u0 + 1, 1)
            start_gather(1)
            wait_gather(0)

            @pl.when(g >= 1)
            def _():
                wait_writes(0)

            build(u0, 0)
            start_write(u0, 0)

            @pl.when(g + 1 < n_units // 2)
            def _():
                prep(u0 + 2, 0)
                start_gather(0)

            wait_gather(1)

            @pl.when(g >= 1)
            def _():
                wait_writes(1)

            build(u0 + 1, 1)
            start_write(u0 + 1, 1)
            return carry

        lax.fori_loop(0, n_units // 2, body, 0)
        wait_writes(0)
        wait_writes(1)

    return gather


def kernel(x, table):
    b, l = x.shape
    v, d = table.shape
    xt = jnp.transpose(x)
    rtab = table.reshape(v // 4, 4 * d)
    outp = _make_gather(b, l, v, d)(xt, rtab)
    return jnp.transpose(outp, (2, 0, 1))
